# baseline (speedup 1.0000x reference)
import functools

import jax
import jax.numpy as jnp
from jax import lax
from jax.experimental import pallas as pl
from jax.experimental.pallas import tpu as pltpu
from jax.experimental.pallas import tpu_sc as plsc

_NC = 2
_NS = 16
_NW = _NC * _NS
_LANES = 16


@functools.lru_cache(maxsize=None)
def _make_pool(B, L, V, D):
    assert D == _LANES and B % 128 == 0
    LPW = 8
    NACT = L // LPW
    assert NACT <= _NW and L % LPW == 0
    NCH = B // 128
    UNR = 8
    mesh = plsc.VectorSubcoreMesh(
        core_axis_name="c", subcore_axis_name="s",
        num_cores=_NC, num_subcores=_NS)

    @functools.partial(
        pl.kernel,
        out_type=(jax.ShapeDtypeStruct((NACT, B, D), jnp.float32),
                  jax.ShapeDtypeStruct((NACT, B), jnp.float32)),
        mesh=mesh,
        scratch_types=[
            pltpu.VMEM((LPW, B), jnp.int32),
            pltpu.VMEM((B, D), jnp.float32),
            pltpu.VMEM((B, D), jnp.float32),
            pltpu.VMEM((B, D), jnp.float32),
            pltpu.VMEM((B,), jnp.float32),
            pltpu.SemaphoreType.DMA,
            pltpu.SemaphoreType.DMA,
        ],
        compiler_params=pltpu.CompilerParams(use_tc_tiling_on_sc=False,
                                             needs_layout_passes=False),
    )
    def pool(xt_hbm, emb_hbm, outp_hbm, outz_hbm,
             idx_v, buf_a, buf_b, part_v, zc_v, sem_a, sem_b):
        wid = lax.axis_index("s") * _NC + lax.axis_index("c")

        @pl.when(wid < NACT)
        def _():
            pltpu.sync_copy(xt_hbm.at[pl.ds(wid * LPW, LPW)], idx_v)

            bufs = (buf_a, buf_b)
            sems = (sem_a, sem_b)

            def fire(l):
                buf, sem = bufs[l % 2], sems[l % 2]
                return [
                    pltpu.async_copy(
                        emb_hbm.at[idx_v.at[l, pl.ds(c * 128, 128)]],
                        buf.at[pl.ds(c * 128, 128)], sem)
                    for c in range(NCH)
                ]

            descs = fire(0)

            def cnt_body(v, carry):
                c = jnp.zeros((_LANES,), jnp.float32)
                one = jnp.ones((_LANES,), jnp.float32)
                zero = jnp.zeros((_LANES,), jnp.float32)
                for l in range(LPW):
                    z = idx_v[l, pl.ds(v * _LANES, _LANES)] == 0
                    c = c + jnp.where(z, one, zero)
                zc_v[pl.ds(v * _LANES, _LANES)] = c
                return carry

            lax.fori_loop(0, B // _LANES, cnt_body, 0)

            for l in range(LPW):
                nxt = fire(l + 1) if l + 1 < LPW else []
                for d in descs:
                    d.wait()
                buf = bufs[l % 2]

                if l == 0:
                    def init_body(v, carry, _buf=buf):
                        j = v * UNR
                        for k in range(UNR):
                            part_v[j + k] = _buf[j + k]
                        return carry

                    lax.fori_loop(0, B // UNR, init_body, 0)
                else:
                    def add_body(v, carry, _buf=buf):
                        j = v * UNR
                        for k in range(UNR):
                            plsc.addupdate(part_v.at[j + k],
                                           _buf[j + k])
                        return carry

                    lax.fori_loop(0, B // UNR, add_body, 0)
                descs = nxt
            pltpu.sync_copy(part_v, outp_hbm.at[wid])
            pltpu.sync_copy(zc_v, outz_hbm.at[wid])

    return pool


@functools.lru_cache(maxsize=None)
def _make_reduce(NACT, B, D):
    def rd(parts_ref, zcs_ref, emb0_ref, out_ref):
        p = jnp.sum(parts_ref[...], axis=0)
        z = jnp.sum(zcs_ref[...], axis=0)
        out_ref[...] = p - z[:, None] * emb0_ref[...]

    return pl.pallas_call(
        rd,
        out_shape=jax.ShapeDtypeStruct((B, D), jnp.float32),
    )


@functools.lru_cache(maxsize=None)
def _make_fc1(B, V, D, vblk=2048):
    grid = (V + vblk - 1) // vblk

    def mm(wt_ref, pooled_ref, b_ref, out_ref):
        out_ref[...] = lax.dot_general(
            wt_ref[...], pooled_ref[...],
            (((0,), (1,)), ((), ())),
            preferred_element_type=jnp.float32,
        ) + b_ref[...].T

    return pl.pallas_call(
        mm,
        grid=(grid,),
        in_specs=[
            pl.BlockSpec((D, vblk), lambda i: (0, i)),
            pl.BlockSpec((B, D), lambda i: (0, 0)),
            pl.BlockSpec((1, vblk), lambda i: (0, i)),
        ],
        out_specs=pl.BlockSpec((vblk, B), lambda i: (i, 0)),
        out_shape=jax.ShapeDtypeStruct((V, B), jnp.float32),
    )


def kernel(x, emb, W, b):
    B, L = x.shape
    V, D = emb.shape
    parts, zcs = _make_pool(B, L, V, D)(x.T, emb)
    pooled = _make_reduce(L // 8, B, D)(parts, zcs, emb[0:1])
    logits_t = _make_fc1(B, V, D)(W.T, pooled, b.reshape(1, V))
    return logits_t.T


# speedup vs baseline: 1.0585x; 1.0585x over previous
import functools

import jax
import jax.numpy as jnp
from jax import lax
from jax.experimental import pallas as pl
from jax.experimental.pallas import tpu as pltpu
from jax.experimental.pallas import tpu_sc as plsc

_NC = 2
_NS = 16
_NW = _NC * _NS
_LANES = 16


@functools.lru_cache(maxsize=None)
def _make_xrepack(B, L):
    nk = B // 128

    def xr(in_ref, out_ref):
        for k in range(nk):
            out_ref[:, k, :] = in_ref[:, k * 128:(k + 1) * 128]

    return pl.pallas_call(
        xr,
        out_shape=jax.ShapeDtypeStruct((L, nk, 128), jnp.int32),
    )


@functools.lru_cache(maxsize=None)
def _make_pool(B, L, V, D):
    assert D == _LANES
    CH0 = 104
    CH1 = L - CH0
    assert CH0 % 8 == 0 and CH1 % 8 == 0 and CH0 <= 128 and CH1 <= 128
    rows_per_w = B // _NW
    RPS = 4
    nsup = rows_per_w // RPS
    sup_rows = RPS * L
    UNR = 8
    assert L % UNR == 0
    nfull = L // _LANES
    rem = L - nfull * _LANES
    mesh = plsc.VectorSubcoreMesh(
        core_axis_name="c", subcore_axis_name="s",
        num_cores=_NC, num_subcores=_NS)

    @functools.partial(
        pl.kernel,
        out_type=jax.ShapeDtypeStruct((B, D), jnp.float32),
        mesh=mesh,
        scratch_types=[
            pltpu.VMEM((L, rows_per_w), jnp.int32),
            pltpu.VMEM((rows_per_w, L), jnp.int32),
            pltpu.VMEM((sup_rows, D), jnp.float32),
            pltpu.VMEM((sup_rows, D), jnp.float32),
            pltpu.VMEM((rows_per_w, D), jnp.float32),
            pltpu.VMEM((rows_per_w, _LANES), jnp.float32),
            pltpu.VMEM((1, D), jnp.float32),
            pltpu.SemaphoreType.DMA,
            pltpu.SemaphoreType.DMA,
        ],
        compiler_params=pltpu.CompilerParams(use_tc_tiling_on_sc=False,
                                             needs_layout_passes=False),
    )
    def pool(x_hbm, emb_hbm, out_hbm,
             raw_v, idx_v, buf_a, buf_b, acc_v, cnt_v, emb0_v,
             sem_a, sem_b):
        wid = lax.axis_index("s") * _NC + lax.axis_index("c")
        col = wid * rows_per_w
        pltpu.sync_copy(
            x_hbm.at[:, col // 128, pl.ds(col % 128, rows_per_w)], raw_v)

        lane = lax.iota(jnp.int32, _LANES)
        ntr = L // _LANES + (1 if L % _LANES else 0)

        def tr_body(r, carry):
            rv = jnp.broadcast_to(r, (_LANES,)).astype(jnp.int32)
            for k in range(ntr):
                base = min(k * _LANES, L - _LANES)
                vals = plsc.load_gather(raw_v, [lane + base, rv])
                idx_v[r, pl.ds(base, _LANES)] = vals
            return carry

        lax.fori_loop(0, rows_per_w, tr_body, 0)
        plsc.subcore_barrier()

        bufs = (buf_a, buf_b)
        sems = (sem_a, sem_b)

        def fire(s):
            buf, sem = bufs[s % 2], sems[s % 2]
            ds = []
            for rr in range(RPS):
                r = s * RPS + rr
                ds.append(pltpu.async_copy(
                    emb_hbm.at[idx_v.at[r, pl.ds(0, CH0)]],
                    buf.at[pl.ds(rr * L, CH0)], sem))
                ds.append(pltpu.async_copy(
                    emb_hbm.at[idx_v.at[r, pl.ds(CH0, CH1)]],
                    buf.at[pl.ds(rr * L + CH0, CH1)], sem))
            return ds

        descs = fire(0)

        pltpu.sync_copy(emb_hbm.at[pl.ds(0, 1)], emb0_v)
        himask = lax.iota(jnp.int32, _LANES) >= (_LANES - rem)

        def cnt_body(r, carry):
            c = jnp.zeros((_LANES,), jnp.int32)
            for k in range(nfull):
                z = idx_v[r, pl.ds(k * _LANES, _LANES)] == 0
                c = c + plsc.all_reduce_population_count(z)
            zt = idx_v[r, pl.ds(L - _LANES, _LANES)] == 0
            c = c + plsc.all_reduce_population_count(
                jnp.logical_and(zt, himask))
            cnt_v[r] = c.astype(jnp.float32)
            return carry

        lax.fori_loop(0, rows_per_w, cnt_body, 0)
        emb0 = emb0_v[0]

        for s in range(nsup):
            nxt = fire(s + 1) if s + 1 < nsup else []
            for d in descs:
                d.wait()
            buf = bufs[s % 2]
            for rr in range(RPS):
                base = rr * L

                def acc_body(v, accs, _base=base, _buf=buf):
                    j = _base + v * UNR
                    return tuple(
                        accs[k] + _buf[j + k] for k in range(UNR))

                accs = lax.fori_loop(
                    0, L // UNR, acc_body,
                    tuple(jnp.zeros((D,), jnp.float32)
                          for _ in range(UNR)))
                acc = accs[0]
                for k in range(1, UNR):
                    acc = acc + accs[k]
                row = s * RPS + rr
                acc_v[row] = acc - cnt_v[row] * emb0
            descs = nxt
        pltpu.sync_copy(acc_v, out_hbm.at[pl.ds(wid * rows_per_w,
                                                rows_per_w)])

    return pool


@functools.lru_cache(maxsize=None)
def _make_fc1(B, V, D, vblk=2048):
    grid = (V + vblk - 1) // vblk

    def mm(wt_ref, pooled_ref, b_ref, out_ref):
        out_ref[...] = lax.dot_general(
            wt_ref[...], pooled_ref[...],
            (((0,), (1,)), ((), ())),
            preferred_element_type=jnp.float32,
        ) + b_ref[...].T

    return pl.pallas_call(
        mm,
        grid=(grid,),
        in_specs=[
            pl.BlockSpec((D, vblk), lambda i: (0, i)),
            pl.BlockSpec((B, D), lambda i: (0, 0)),
            pl.BlockSpec((1, vblk), lambda i: (0, i)),
        ],
        out_specs=pl.BlockSpec((vblk, B), lambda i: (i, 0)),
        out_shape=jax.ShapeDtypeStruct((V, B), jnp.float32),
    )


def kernel(x, emb, W, b):
    B, L = x.shape
    V, D = emb.shape
    x3 = _make_xrepack(B, L)(x.T.astype(jnp.int32))
    pooled = _make_pool(B, L, V, D)(x3, emb)
    logits_t = _make_fc1(B, V, D)(W.T, pooled, b.reshape(1, V))
    return logits_t.T
